# Initial kernel scaffold; baseline (speedup 1.0000x reference)
import jax
import jax.numpy as jnp
from jax.experimental import pallas as pl


def kernel(token_ids, weight):
    raise NotImplementedError("write your pallas kernel here")


# speedup vs baseline: 2.9740x; 2.9740x over previous
import functools

import jax
import jax.numpy as jnp
from jax import lax
from jax.experimental import pallas as pl
from jax.experimental.pallas import tpu as pltpu
from jax.experimental.pallas import tpu_sc as plsc

_CHUNK = 128


def _sc_geometry():
    try:
        info = plsc.get_sparse_core_info()
        return info.num_cores, info.num_subcores
    except Exception:
        return 2, 16


@functools.lru_cache(maxsize=None)
def _make_gather(B, D, NC, NS):
    NW = NC * NS
    b_per_w = B // NW
    n_chunks = b_per_w // _CHUNK
    mesh = plsc.VectorSubcoreMesh(core_axis_name="c", subcore_axis_name="s")

    @functools.partial(
        pl.kernel,
        out_type=jax.ShapeDtypeStruct((B, D), jnp.float32),
        mesh=mesh,
        scratch_types=[
            pltpu.VMEM((n_chunks, _CHUNK), jnp.int32),
            pltpu.VMEM((_CHUNK, D), jnp.float32),
            pltpu.SemaphoreType.DMA,
        ],
    )
    def gather_kernel(table_hbm, idx_hbm, out_hbm, idx_v, rows_v, sem):
        wid = lax.axis_index("s") * NC + lax.axis_index("c")
        base = wid * b_per_w
        pltpu.sync_copy(idx_hbm.at[wid], idx_v)

        def step(c, carry):
            pltpu.async_copy(table_hbm.at[idx_v.at[c]], rows_v, sem).wait()
            pltpu.sync_copy(rows_v, out_hbm.at[pl.ds(base + c * _CHUNK, _CHUNK)])
            return carry

        lax.fori_loop(0, n_chunks, step, 0)

    return gather_kernel


def kernel(token_ids, weight):
    B0, S = token_ids.shape
    D = weight.shape[1]
    B = B0 * S
    NC, NS = _sc_geometry()
    idx3 = token_ids.reshape(NC * NS, -1, _CHUNK).astype(jnp.int32)
    out = _make_gather(B, D, NC, NS)(weight, idx3)
    return out.reshape(B0, S, D)


# speedup vs baseline: 3.1333x; 1.0536x over previous
import functools

import jax
import jax.numpy as jnp
from jax import lax
from jax.experimental import pallas as pl
from jax.experimental.pallas import tpu as pltpu
from jax.experimental.pallas import tpu_sc as plsc

_CHUNK = 128


def _sc_geometry():
    try:
        info = plsc.get_sparse_core_info()
        return info.num_cores, info.num_subcores
    except Exception:
        return 2, 16


@functools.lru_cache(maxsize=None)
def _make_gather(B, D, NC, NS):
    NW = NC * NS
    b_per_w = B // NW
    n_chunks = b_per_w // _CHUNK
    mesh = plsc.VectorSubcoreMesh(core_axis_name="c", subcore_axis_name="s")

    n2 = n_chunks // 2
    assert n_chunks == 2 * n2 and n2 >= 2

    @functools.partial(
        pl.kernel,
        out_type=jax.ShapeDtypeStruct((B, D), jnp.float32),
        mesh=mesh,
        scratch_types=[
            pltpu.VMEM((n_chunks, _CHUNK), jnp.int32),
            pltpu.VMEM((2, _CHUNK, D), jnp.float32),
            pltpu.SemaphoreType.DMA,
            pltpu.SemaphoreType.DMA,
            pltpu.SemaphoreType.DMA,
            pltpu.SemaphoreType.DMA,
        ],
    )
    def gather_kernel(table_hbm, idx_hbm, out_hbm, idx_v, rows_v,
                      gsem0, gsem1, ssem0, ssem1):
        wid = lax.axis_index("s") * NC + lax.axis_index("c")
        base = wid * b_per_w
        pltpu.sync_copy(idx_hbm.at[wid], idx_v)

        def gather(c, b, sem):
            return pltpu.make_async_copy(
                table_hbm.at[idx_v.at[c]], rows_v.at[b], sem)

        def store(c, b, sem):
            return pltpu.make_async_copy(
                rows_v.at[b], out_hbm.at[pl.ds(base + c * _CHUNK, _CHUNK)], sem)

        gather(0, 0, gsem0).start()

        def step(c2, carry):
            c = 2 * c2
            gather(c, 0, gsem0).wait()

            @pl.when(c2 >= 1)
            def _():
                store(c - 1, 1, ssem1).wait()

            gather(c + 1, 1, gsem1).start()
            store(c, 0, ssem0).start()
            gather(c + 1, 1, gsem1).wait()

            @pl.when(c2 < n2 - 1)
            def _():
                store(c, 0, ssem0).wait()
                gather(c + 2, 0, gsem0).start()

            store(c + 1, 1, ssem1).start()
            return carry

        lax.fori_loop(0, n2, step, 0)
        store(n_chunks - 2, 0, ssem0).wait()
        store(n_chunks - 1, 1, ssem1).wait()

    return gather_kernel


def kernel(token_ids, weight):
    B0, S = token_ids.shape
    D = weight.shape[1]
    B = B0 * S
    NC, NS = _sc_geometry()
    idx3 = token_ids.reshape(NC * NS, -1, _CHUNK).astype(jnp.int32)
    out = _make_gather(B, D, NC, NS)(weight, idx3)
    return out.reshape(B0, S, D)


# speedup vs baseline: 3.3429x; 1.0669x over previous
import functools

import jax
import jax.numpy as jnp
from jax import lax
from jax.experimental import pallas as pl
from jax.experimental.pallas import tpu as pltpu
from jax.experimental.pallas import tpu_sc as plsc

_CHUNK = 128


def _sc_geometry():
    try:
        info = plsc.get_sparse_core_info()
        return info.num_cores, info.num_subcores
    except Exception:
        return 2, 16


@functools.lru_cache(maxsize=None)
def _make_gather(B, D, NC, NS):
    NW = NC * NS
    b_per_w = B // NW
    n_chunks = b_per_w // _CHUNK
    mesh = plsc.VectorSubcoreMesh(core_axis_name="c", subcore_axis_name="s")

    NBUF = 5
    n_steps = n_chunks // NBUF
    assert n_chunks == NBUF * n_steps and n_steps >= 2

    @functools.partial(
        pl.kernel,
        out_type=jax.ShapeDtypeStruct((B, D), jnp.float32),
        mesh=mesh,
        scratch_types=[
            pltpu.VMEM((n_chunks, _CHUNK), jnp.int32),
            pltpu.VMEM((NBUF, _CHUNK, D), jnp.float32),
            [pltpu.SemaphoreType.DMA] * NBUF,
            [pltpu.SemaphoreType.DMA] * NBUF,
        ],
    )
    def gather_kernel(table_hbm, idx_hbm, out_hbm, idx_v, rows_v,
                      gsems, ssems):
        wid = lax.axis_index("s") * NC + lax.axis_index("c")
        base = wid * b_per_w
        pltpu.sync_copy(idx_hbm.at[wid], idx_v)

        def gather(c, b):
            return pltpu.make_async_copy(
                table_hbm.at[idx_v.at[c]], rows_v.at[b], gsems[b])

        def store(c, b):
            return pltpu.make_async_copy(
                rows_v.at[b], out_hbm.at[pl.ds(base + c * _CHUNK, _CHUNK)],
                ssems[b])

        for j in range(NBUF):
            gather(j, j).start()

        def step(i, carry):
            c0 = i * NBUF
            for j in range(NBUF):
                c = c0 + j
                gather(c, j).wait()
                store(c, j).start()

                @pl.when(i < n_steps - 1)
                def _(c=c, j=j):
                    store(c, j).wait()
                    gather(c + NBUF, j).start()

            return carry

        lax.fori_loop(0, n_steps, step, 0)
        for j in range(NBUF):
            store(n_chunks - NBUF + j, j).wait()

    return gather_kernel


def kernel(token_ids, weight):
    B0, S = token_ids.shape
    D = weight.shape[1]
    B = B0 * S
    NC, NS = _sc_geometry()
    idx3 = token_ids.reshape(NC * NS, -1, _CHUNK).astype(jnp.int32)
    out = _make_gather(B, D, NC, NS)(weight, idx3)
    return out.reshape(B0, S, D)
